# baseline (speedup 1.0000x reference)
import functools
import math

import jax
import jax.numpy as jnp
from jax import lax
from jax.experimental import pallas as pl
from jax.experimental.pallas import tpu as pltpu
from jax.experimental.pallas import tpu_sc as plsc

D_MODEL = 64
SCALE = math.sqrt(D_MODEL)

_info = plsc.get_sparse_core_info()
NC, NS, L = _info.num_cores, _info.num_subcores, _info.num_lanes
NW = NC * NS

CHUNK = 128


@functools.partial(jax.jit, static_argnames=("n_tokens",))
def _embed_gather(table, idx, *, n_tokens):
    b_per_w = n_tokens // NW
    n_chunks = b_per_w // CHUNK
    mesh = plsc.VectorSubcoreMesh(core_axis_name="c", subcore_axis_name="s")

    @functools.partial(
        pl.kernel,
        mesh=mesh,
        out_type=jax.ShapeDtypeStruct((n_tokens, D_MODEL), jnp.float32),
        scratch_types=[
            pltpu.VMEM((CHUNK,), jnp.int32),
            pltpu.VMEM((CHUNK, D_MODEL), jnp.float32),
            pltpu.SemaphoreType.DMA,
        ],
        compiler_params=pltpu.CompilerParams(use_tc_tiling_on_sc=False),
    )
    def k(table_hbm, idx_hbm, out_hbm, idx_v, rows_v, sem):
        wid = lax.axis_index("s") * NC + lax.axis_index("c")
        base = wid * b_per_w

        def chunk_body(g, carry):
            off = base + g * CHUNK
            pltpu.sync_copy(idx_hbm.at[pl.ds(off, CHUNK)], idx_v)
            pltpu.async_copy(table_hbm.at[idx_v], rows_v, sem).wait()

            def scale_row(r, c2):
                for c4 in range(D_MODEL // L):
                    sl = rows_v[r, pl.ds(c4 * L, L)]
                    rows_v[r, pl.ds(c4 * L, L)] = sl * SCALE
                return c2

            lax.fori_loop(0, CHUNK, scale_row, 0)
            pltpu.sync_copy(rows_v, out_hbm.at[pl.ds(off, CHUNK)])
            return carry

        lax.fori_loop(0, n_chunks, chunk_body, 0)

    return k(table, idx)


def kernel(x, embed_weight):
    s0, s1 = x.shape
    n_tokens = s0 * s1
    idx = x.reshape(n_tokens).astype(jnp.int32)
    out = _embed_gather(embed_weight, idx, n_tokens=n_tokens)
    return out.reshape(s0, s1, D_MODEL)


# speedup vs baseline: 1.2482x; 1.2482x over previous
import functools
import math

import jax
import jax.numpy as jnp
from jax import lax
from jax.experimental import pallas as pl
from jax.experimental.pallas import tpu as pltpu
from jax.experimental.pallas import tpu_sc as plsc

D_MODEL = 64
SCALE = math.sqrt(D_MODEL)

_info = plsc.get_sparse_core_info()
NC, NS, L = _info.num_cores, _info.num_subcores, _info.num_lanes
NW = NC * NS

CHUNK = 128
NBUF = 4
SLICES = CHUNK * D_MODEL // L


@functools.partial(jax.jit, static_argnames=("n_tokens",))
def _embed_gather(table, idx, *, n_tokens):
    b_per_w = n_tokens // NW
    n_chunks = b_per_w // CHUNK
    n_grp = n_chunks // NBUF
    mesh = plsc.VectorSubcoreMesh(core_axis_name="c", subcore_axis_name="s")

    @functools.partial(
        pl.kernel,
        mesh=mesh,
        out_type=jax.ShapeDtypeStruct((n_tokens, D_MODEL), jnp.float32),
        scratch_types=(
            [pltpu.VMEM((b_per_w,), jnp.int32)]
            + [pltpu.VMEM((CHUNK, D_MODEL), jnp.float32)] * NBUF
            + [pltpu.SemaphoreType.DMA] * (2 * NBUF)
        ),
        compiler_params=pltpu.CompilerParams(use_tc_tiling_on_sc=False),
    )
    def k(table_hbm, idx_hbm, out_hbm, idx_all, r0, r1, r2, r3, *sems):
        rows = [r0, r1, r2, r3]
        sem_g = sems[:NBUF]
        sem_o = sems[NBUF:]
        wid = lax.axis_index("s") * NC + lax.axis_index("c")
        base = wid * b_per_w

        pltpu.sync_copy(idx_hbm.at[pl.ds(base, b_per_w)], idx_all)

        def start_gather(g, b):
            pltpu.async_copy(
                table_hbm.at[idx_all.at[pl.ds(g * CHUNK, CHUNK)]],
                rows[b], sem_g[b])

        def wait_gather(b):
            pltpu.make_async_copy(
                table_hbm.at[idx_all.at[pl.ds(0, CHUNK)]],
                rows[b], sem_g[b]).wait()

        def start_out(g, b):
            pltpu.async_copy(
                rows[b], out_hbm.at[pl.ds(base + g * CHUNK, CHUNK)], sem_o[b])

        def wait_out(b):
            pltpu.make_async_copy(
                rows[b], out_hbm.at[pl.ds(base, CHUNK)], sem_o[b]).wait()

        def scale(b):
            rb = rows[b]

            @plsc.parallel_loop(0, SLICES, unroll=8)
            def _(i):
                r = lax.shift_right_logical(i, 2)
                c = lax.shift_left(lax.bitwise_and(i, 3), 4)
                rb[r, pl.ds(c, L)] = rb[r, pl.ds(c, L)] * SCALE

        start_gather(0, 0)
        start_gather(1, 1)

        for b in range(NBUF):
            g = b
            wait_gather(b)
            scale(b)
            start_out(g, b)
            b2 = (b + 2) % NBUF
            if b >= 2:
                wait_out(b2)
            start_gather(g + 2, b2)

        def grp_body(grp, carry):
            g0 = grp * NBUF
            for b in range(NBUF):
                g = g0 + b
                wait_gather(b)
                scale(b)
                start_out(g, b)
                b2 = (b + 2) % NBUF
                wait_out(b2)
                start_gather(g + 2, b2)
            return carry

        lax.fori_loop(1, n_grp - 1, grp_body, 0)

        g0 = (n_grp - 1) * NBUF
        for b in range(NBUF):
            g = g0 + b
            wait_gather(b)
            scale(b)
            start_out(g, b)
            b2 = (b + 2) % NBUF
            if b < 2:
                wait_out(b2)
                start_gather(g + 2, b2)
        for b in range(NBUF):
            wait_out(b)

    return k(table, idx)


def kernel(x, embed_weight):
    s0, s1 = x.shape
    n_tokens = s0 * s1
    idx = x.reshape(n_tokens).astype(jnp.int32)
    out = _embed_gather(embed_weight, idx, n_tokens=n_tokens)
    return out.reshape(s0, s1, D_MODEL)
